# baseline (speedup 1.0000x reference)
import functools

import jax
import jax.numpy as jnp
from jax import lax
from jax.experimental import pallas as pl
from jax.experimental.pallas import tpu as pltpu
from jax.experimental.pallas import tpu_sc as plsc

BATCH = 4096
DIM = 128
NCHUNK = 4


def _build():
    info = plsc.get_sparse_core_info()
    nc, ns, lanes = info.num_cores, info.num_subcores, info.num_lanes
    nw = nc * ns
    bpw = BATCH // nw

    mesh = plsc.VectorSubcoreMesh(core_axis_name="c", subcore_axis_name="s")

    @functools.partial(
        pl.kernel,
        mesh=mesh,
        out_type=jax.ShapeDtypeStruct((BATCH, DIM), jnp.float32),
        scratch_types=[
            pltpu.VMEM((bpw,), jnp.int32),
            pltpu.VMEM((bpw, DIM), jnp.float32),
            pltpu.SemaphoreType.DMA,
        ]
        + [pltpu.SemaphoreType.DMA] * (3 * NCHUNK),
    )
    def sc_kernel(x_hbm, idx_hbm, delta_hbm, out_hbm, idx_v, x_v, semi, *sems):
        wid = lax.axis_index("s") * nc + lax.axis_index("c")
        base = wid * bpw
        ck = bpw // NCHUNK
        semx = sems[:NCHUNK]
        semg = sems[NCHUNK : 2 * NCHUNK]
        semo = sems[2 * NCHUNK :]
        ci = pltpu.async_copy(idx_hbm.at[pl.ds(base, bpw)], idx_v, semi)
        cx = [
            pltpu.async_copy(
                x_hbm.at[pl.ds(base + c * ck, ck)],
                x_v.at[pl.ds(c * ck, ck)],
                semx[c],
            )
            for c in range(NCHUNK)
        ]
        ci.wait()
        g = []
        for c in range(NCHUNK):
            cx[c].wait()
            g.append(
                pltpu.async_copy(
                    delta_hbm.at[idx_v.at[pl.ds(c * ck, ck)]],
                    x_v.at[pl.ds(c * ck, ck)],
                    semg[c],
                    add=True,
                )
            )
        o = []
        for c in range(NCHUNK):
            g[c].wait()
            o.append(
                pltpu.async_copy(
                    x_v.at[pl.ds(c * ck, ck)],
                    out_hbm.at[pl.ds(base + c * ck, ck)],
                    semo[c],
                )
            )
        for c in range(NCHUNK):
            o[c].wait()

    return sc_kernel


_sc_kernel = _build()


@jax.jit
def kernel(x, idx, delta):
    return _sc_kernel(x, idx.astype(jnp.int32), delta)


# speedup vs baseline: 1.0062x; 1.0062x over previous
import functools

import jax
import jax.numpy as jnp
from jax import lax
from jax.experimental import pallas as pl
from jax.experimental.pallas import tpu as pltpu
from jax.experimental.pallas import tpu_sc as plsc

BATCH = 4096
DIM = 128
NCHUNK = 2


def _build():
    info = plsc.get_sparse_core_info()
    nc, ns, lanes = info.num_cores, info.num_subcores, info.num_lanes
    nw = nc * ns
    bpw = BATCH // nw

    mesh = plsc.VectorSubcoreMesh(core_axis_name="c", subcore_axis_name="s")

    @functools.partial(
        pl.kernel,
        mesh=mesh,
        out_type=jax.ShapeDtypeStruct((BATCH, DIM), jnp.float32),
        scratch_types=[
            pltpu.VMEM((bpw,), jnp.int32),
            pltpu.VMEM((bpw, DIM), jnp.float32),
            pltpu.SemaphoreType.DMA,
        ]
        + [pltpu.SemaphoreType.DMA] * (3 * NCHUNK),
    )
    def sc_kernel(x_hbm, idx_hbm, delta_hbm, out_hbm, idx_v, x_v, semi, *sems):
        wid = lax.axis_index("s") * nc + lax.axis_index("c")
        base = wid * bpw
        ck = bpw // NCHUNK
        semx = sems[:NCHUNK]
        semg = sems[NCHUNK : 2 * NCHUNK]
        semo = sems[2 * NCHUNK :]
        ci = pltpu.async_copy(idx_hbm.at[pl.ds(base, bpw)], idx_v, semi)
        cx = [
            pltpu.async_copy(
                x_hbm.at[pl.ds(base + c * ck, ck)],
                x_v.at[pl.ds(c * ck, ck)],
                semx[c],
            )
            for c in range(NCHUNK)
        ]
        ci.wait()
        g = []
        for c in range(NCHUNK):
            cx[c].wait()
            g.append(
                pltpu.async_copy(
                    delta_hbm.at[idx_v.at[pl.ds(c * ck, ck)]],
                    x_v.at[pl.ds(c * ck, ck)],
                    semg[c],
                    add=True,
                )
            )
        o = []
        for c in range(NCHUNK):
            g[c].wait()
            o.append(
                pltpu.async_copy(
                    x_v.at[pl.ds(c * ck, ck)],
                    out_hbm.at[pl.ds(base + c * ck, ck)],
                    semo[c],
                )
            )
        for c in range(NCHUNK):
            o[c].wait()

    return sc_kernel


_sc_kernel = _build()


@jax.jit
def kernel(x, idx, delta):
    return _sc_kernel(x, idx.astype(jnp.int32), delta)
